# baseline (speedup 1.0000x reference)
import jax
import jax.numpy as jnp
from jax import lax
from jax.experimental import pallas as pl
from jax.experimental.pallas import tpu as pltpu

B = 64
EDD = 2048
EDS = 1024
CHUNK = 256
NCHUNK = EDS // CHUNK
BN_EPS = 1e-5
COS_EPS = 1e-8

_DN_T = (((1,), (1,)), ((), ()))
_DN = (((1,), (0,)), ((), ()))


def _fused_kernel(x_ref, y_ref, w_hbm, out_ref, wbuf, sems):
    copies = []
    for k in range(NCHUNK):
        c = pltpu.make_async_copy(
            w_hbm.at[pl.ds(k * CHUNK, CHUNK), :], wbuf.at[k], sems.at[k])
        c.start()
        copies.append(c)

    ones_row = jnp.ones((1, B), dtype=jnp.float32)
    ones_col = jnp.ones((CHUNK, 1), dtype=jnp.float32)
    lane = lax.broadcasted_iota(jnp.int32, (B, CHUNK), 1)
    at_block_start = (lane % 4) == 0
    low = jnp.full((B, CHUNK), -2.0, dtype=jnp.float32)

    def bn_tanh(hh):
        s1 = lax.dot_general(ones_row, hh, _DN,
                             preferred_element_type=jnp.float32)
        s2 = lax.dot_general(ones_row, hh * hh, _DN,
                             preferred_element_type=jnp.float32)
        mu = s1 * (1.0 / B)
        var = s2 * (1.0 / B) - mu * mu
        scale = lax.rsqrt(var + BN_EPS)
        return jnp.tanh((hh - mu) * scale)

    def block_mask(hh):
        a = jnp.maximum(hh, pltpu.roll(hh, CHUNK - 1, 1))
        bm = jnp.maximum(a, pltpu.roll(a, CHUNK - 2, 1))
        c = jnp.where(at_block_start, bm, low)
        c = jnp.maximum(c, pltpu.roll(c, 1, 1))
        bmax = jnp.maximum(c, pltpu.roll(c, 2, 1))
        return jnp.where(hh == bmax, hh, 0.0)

    dot = jnp.zeros((B, 1), dtype=jnp.float32)
    nx = jnp.zeros((B, 1), dtype=jnp.float32)
    ny = jnp.zeros((B, 1), dtype=jnp.float32)
    for k in range(NCHUNK):
        copies[k].wait()
        w = wbuf[k]
        hx = lax.dot_general(x_ref[...], w, _DN_T,
                             preferred_element_type=jnp.float32)
        hy = lax.dot_general(y_ref[...], w, _DN_T,
                             preferred_element_type=jnp.float32)
        mx = block_mask(bn_tanh(hx))
        my = block_mask(bn_tanh(hy))
        dot += lax.dot_general(mx * my, ones_col, _DN,
                               preferred_element_type=jnp.float32)
        nx += lax.dot_general(mx * mx, ones_col, _DN,
                              preferred_element_type=jnp.float32)
        ny += lax.dot_general(my * my, ones_col, _DN,
                              preferred_element_type=jnp.float32)

    nxc = jnp.maximum(jnp.sqrt(nx), COS_EPS)
    nyc = jnp.maximum(jnp.sqrt(ny), COS_EPS)
    out_ref[...] = (dot / (nxc * nyc)).reshape(B)


def kernel(x, y, W, b, gamma_x, beta_x, gamma_y, beta_y):
    out = pl.pallas_call(
        _fused_kernel,
        in_specs=[
            pl.BlockSpec((B, EDD), lambda: (0, 0)),
            pl.BlockSpec((B, EDD), lambda: (0, 0)),
            pl.BlockSpec(memory_space=pltpu.MemorySpace.HBM),
        ],
        out_specs=pl.BlockSpec((B,), lambda: (0,)),
        out_shape=jax.ShapeDtypeStruct((B,), jnp.float32),
        scratch_shapes=[
            pltpu.VMEM((NCHUNK, CHUNK, EDD), jnp.float32),
            pltpu.SemaphoreType.DMA((NCHUNK,)),
        ],
    )(x, y, W)
    return out


# speedup vs baseline: 1.0670x; 1.0670x over previous
import jax
import jax.numpy as jnp
from jax import lax
from jax.experimental import pallas as pl
from jax.experimental.pallas import tpu as pltpu

B = 64
EDD = 2048
EDS = 1024
CHUNK = 256
NCHUNK = EDS // CHUNK
BN_EPS = 1e-5
COS_EPS = 1e-8

_DN_T = (((1,), (1,)), ((), ()))
_DN = (((1,), (0,)), ((), ()))


def _fused_kernel(x_ref, y_ref, w_ref, out_ref, dot_acc, nx_acc, ny_acc):
    k = pl.program_id(0)

    ones_row = jnp.ones((1, B), dtype=jnp.float32)
    ones_col = jnp.ones((CHUNK, 1), dtype=jnp.float32)
    lane = lax.broadcasted_iota(jnp.int32, (B, CHUNK), 1)
    at_block_start = (lane % 4) == 0
    low = jnp.full((B, CHUNK), -2.0, dtype=jnp.float32)

    def bn_tanh(hh):
        s1 = lax.dot_general(ones_row, hh, _DN,
                             preferred_element_type=jnp.float32)
        s2 = lax.dot_general(ones_row, hh * hh, _DN,
                             preferred_element_type=jnp.float32)
        mu = s1 * (1.0 / B)
        var = s2 * (1.0 / B) - mu * mu
        scale = lax.rsqrt(var + BN_EPS)
        return jnp.tanh((hh - mu) * scale)

    def block_mask(hh):
        a = jnp.maximum(hh, pltpu.roll(hh, CHUNK - 1, 1))
        bm = jnp.maximum(a, pltpu.roll(a, CHUNK - 2, 1))
        c = jnp.where(at_block_start, bm, low)
        c = jnp.maximum(c, pltpu.roll(c, 1, 1))
        bmax = jnp.maximum(c, pltpu.roll(c, 2, 1))
        return jnp.where(hh == bmax, hh, 0.0)

    w = w_ref[...]
    hx = lax.dot_general(x_ref[...], w, _DN_T,
                         preferred_element_type=jnp.float32)
    hy = lax.dot_general(y_ref[...], w, _DN_T,
                         preferred_element_type=jnp.float32)
    mx = block_mask(bn_tanh(hx))
    my = block_mask(bn_tanh(hy))
    dot = lax.dot_general(mx * my, ones_col, _DN,
                          preferred_element_type=jnp.float32)
    nx = lax.dot_general(mx * mx, ones_col, _DN,
                         preferred_element_type=jnp.float32)
    ny = lax.dot_general(my * my, ones_col, _DN,
                         preferred_element_type=jnp.float32)

    @pl.when(k == 0)
    def _():
        dot_acc[...] = dot
        nx_acc[...] = nx
        ny_acc[...] = ny

    @pl.when(k > 0)
    def _():
        dot_acc[...] += dot
        nx_acc[...] += nx
        ny_acc[...] += ny

    @pl.when(k == NCHUNK - 1)
    def _():
        nxc = jnp.maximum(jnp.sqrt(nx_acc[...]), COS_EPS)
        nyc = jnp.maximum(jnp.sqrt(ny_acc[...]), COS_EPS)
        out_ref[...] = (dot_acc[...] / (nxc * nyc)).reshape(B)


def kernel(x, y, W, b, gamma_x, beta_x, gamma_y, beta_y):
    out = pl.pallas_call(
        _fused_kernel,
        grid=(NCHUNK,),
        in_specs=[
            pl.BlockSpec((B, EDD), lambda k: (0, 0)),
            pl.BlockSpec((B, EDD), lambda k: (0, 0)),
            pl.BlockSpec((CHUNK, EDD), lambda k: (k, 0)),
        ],
        out_specs=pl.BlockSpec((B,), lambda k: (0,)),
        out_shape=jax.ShapeDtypeStruct((B,), jnp.float32),
        scratch_shapes=[
            pltpu.VMEM((B, 1), jnp.float32),
            pltpu.VMEM((B, 1), jnp.float32),
            pltpu.VMEM((B, 1), jnp.float32),
        ],
    )(x, y, W)
    return out
